# Initial kernel scaffold; baseline (speedup 1.0000x reference)
import functools

import jax
import jax.numpy as jnp
from jax import lax
from jax.experimental import pallas as pl
from jax.experimental.pallas import tpu as pltpu
from jax.experimental.pallas import tpu_sc as plsc

NC = 2
NS = 16
NW = NC * NS
G = 128
LANES = 16


def _make_sc_agg(with_deg, n_pad, d, K):
    RT = n_pad // NS

    def body(*refs):
        if with_deg:
            (h_hbm, srcw_hbm, dstw_hbm, agg_hbm, deg_hbm,
             src_v, dst_v, gbuf, ones_v, zdeg_v, agg_sp, deg_sp, gsem) = refs
        else:
            (h_hbm, srcw_hbm, dstw_hbm, agg_hbm,
             src_v, dst_v, gbuf, agg_sp, gsem) = refs
        c = lax.axis_index("c")
        s = lax.axis_index("s")
        w = c * NS + s

        pltpu.sync_copy(srcw_hbm.at[w], src_v)
        pltpu.sync_copy(dstw_hbm.at[w], dst_v)

        def zrow(r, carry):
            for j in range(d // LANES):
                gbuf[0, r, pl.ds(j * LANES, LANES)] = jnp.zeros((LANES,), jnp.float32)
            return carry
        lax.fori_loop(0, G, zrow, 0)
        for t in range(RT // G):
            pltpu.sync_copy(gbuf.at[0], agg_sp.at[pl.ds(s * RT + t * G, G)])
        if with_deg:
            def zdrow(r, carry):
                zdeg_v[pl.ds(r * LANES, LANES)] = jnp.zeros((LANES,), jnp.float32)
                return carry
            lax.fori_loop(0, RT // LANES, zdrow, 0)
            pltpu.sync_copy(zdeg_v, deg_sp.at[pl.ds(s * RT, RT)])

            def orow(r, carry):
                ones_v[pl.ds(r * LANES, LANES)] = jnp.ones((LANES,), jnp.float32)
                return carry
            lax.fori_loop(0, G // LANES, orow, 0)
        plsc.subcore_barrier()

        pltpu.async_copy(h_hbm.at[src_v.at[0]], gbuf.at[0], gsem)

        def step(i, carry):
            pltpu.async_copy(h_hbm.at[src_v.at[i + 1]],
                             gbuf.at[lax.rem(i + 1, 2)], gsem)
            pltpu.make_async_copy(h_hbm.at[src_v.at[i]],
                                  gbuf.at[lax.rem(i, 2)], gsem).wait()
            pltpu.sync_copy(gbuf.at[lax.rem(i, 2)],
                            agg_sp.at[dst_v.at[i]], add=True)
            if with_deg:
                pltpu.sync_copy(ones_v, deg_sp.at[dst_v.at[i]], add=True)
            return carry
        lax.fori_loop(0, K - 1, step, 0)
        last = K - 1
        pltpu.make_async_copy(h_hbm.at[src_v.at[last]],
                              gbuf.at[lax.rem(last, 2)], gsem).wait()
        pltpu.sync_copy(gbuf.at[lax.rem(last, 2)],
                        agg_sp.at[dst_v.at[last]], add=True)
        if with_deg:
            pltpu.sync_copy(ones_v, deg_sp.at[dst_v.at[last]], add=True)

        plsc.subcore_barrier()
        pltpu.sync_copy(agg_sp.at[pl.ds(s * RT, RT)],
                        agg_hbm.at[c, pl.ds(s * RT, RT)])
        if with_deg:
            pltpu.sync_copy(deg_sp.at[pl.ds(s * RT, RT)],
                            deg_hbm.at[c, pl.ds(s * RT, RT)])

    out_type = [jax.ShapeDtypeStruct((NC, n_pad, d), jnp.float32)]
    scratch = [
        pltpu.VMEM((K, G), jnp.int32),
        pltpu.VMEM((K, G), jnp.int32),
        pltpu.VMEM((2, G, d), jnp.float32),
    ]
    if with_deg:
        out_type.append(jax.ShapeDtypeStruct((NC, n_pad), jnp.float32))
        scratch.append(pltpu.VMEM((G,), jnp.float32))
        scratch.append(pltpu.VMEM((RT,), jnp.float32))
    scratch.append(pltpu.VMEM_SHARED((n_pad, d), jnp.float32))
    if with_deg:
        scratch.append(pltpu.VMEM_SHARED((n_pad,), jnp.float32))
    scratch.append(pltpu.SemaphoreType.DMA)

    mesh = plsc.VectorSubcoreMesh(core_axis_name="c", subcore_axis_name="s")
    return pl.kernel(body, out_type=out_type, mesh=mesh, scratch_types=scratch)


def _tc_combine(h_pad, agg, deg3, wst, wnt, brow, relu, n_pad, d):
    BN = 2048
    grid = (n_pad // BN,)

    def body(h_ref, agg_ref, deg_ref, ws_ref, wn_ref, b_ref, o_ref):
        deg = deg_ref[0] + deg_ref[1]
        scale = 1.0 / jnp.maximum(deg, 1.0)
        hn = (agg_ref[0] + agg_ref[1]) * scale
        acc = (jnp.dot(h_ref[...], ws_ref[...], preferred_element_type=jnp.float32)
               + jnp.dot(hn, wn_ref[...], preferred_element_type=jnp.float32)
               + b_ref[...])
        if relu:
            acc = jnp.maximum(acc, 0.0)
        o_ref[...] = acc

    return pl.pallas_call(
        body,
        grid=grid,
        in_specs=[
            pl.BlockSpec((BN, d), lambda i: (i, 0)),
            pl.BlockSpec((NC, BN, d), lambda i: (0, i, 0)),
            pl.BlockSpec((NC, BN, 1), lambda i: (0, i, 0)),
            pl.BlockSpec((d, d), lambda i: (0, 0)),
            pl.BlockSpec((d, d), lambda i: (0, 0)),
            pl.BlockSpec((1, d), lambda i: (0, 0)),
        ],
        out_specs=pl.BlockSpec((BN, d), lambda i: (i, 0)),
        out_shape=jax.ShapeDtypeStruct((n_pad, d), jnp.float32),
    )(h_pad, agg, deg3, wst, wnt, brow)


def kernel(inputs, edge_index, W_self1, W_neigh1, b1, W_self2, W_neigh2, b2):
    n, d = inputs.shape
    e = edge_index.shape[1]
    chunk_rows = NS * G
    n_pad = ((n + chunk_rows - 1) // chunk_rows) * chunk_rows
    ew = ((e + NW * G - 1) // (NW * G)) * G
    K = ew // G
    e_pad = ew * NW

    src = edge_index[0]
    dst = edge_index[1]
    if e_pad > e:
        pad_dst = min(n, n_pad - 1)
        src = jnp.concatenate([src, jnp.zeros((e_pad - e,), jnp.int32)])
        dst = jnp.concatenate([dst, jnp.full((e_pad - e,), pad_dst, jnp.int32)])
    srcw = src.reshape(NW, K, G)
    dstw = dst.reshape(NW, K, G)

    h0 = inputs
    if n_pad > n:
        h0 = jnp.concatenate([inputs, jnp.zeros((n_pad - n, d), jnp.float32)])

    sc_agg_deg = _make_sc_agg(True, n_pad, d, K)
    sc_agg = _make_sc_agg(False, n_pad, d, K)

    agg1, deg = sc_agg_deg(h0, srcw, dstw)
    deg3 = deg[..., None]
    h1 = _tc_combine(h0, agg1, deg3, W_self1.T, W_neigh1.T, b1[None, :],
                     True, n_pad, d)
    (agg2,) = sc_agg(h1, srcw, dstw)
    out = _tc_combine(h1, agg2, deg3, W_self2.T, W_neigh2.T, b2[None, :],
                      False, n_pad, d)
    return out[:n]


# speedup vs baseline: 4.7817x; 4.7817x over previous
import jax
import jax.numpy as jnp
from jax import lax
from jax.experimental import pallas as pl
from jax.experimental.pallas import tpu as pltpu
from jax.experimental.pallas import tpu_sc as plsc

NC = 2
NS = 16
NW = NC * NS
G = 128
GZ = 64
CW = 32
LANES = 16


def _make_sc_agg(with_deg, n_pad, d, K):
    RT = n_pad // NS
    P = d // CW

    def body(*refs):
        h_hbms = refs[0:P]
        srcw_hbm, dstw_hbm = refs[P], refs[P + 1]
        agg_hbms = refs[P + 2:2 * P + 2]
        k = 2 * P + 2
        if with_deg:
            deg_hbm = refs[k]
            k += 1
        src_v, dst_v, gbuf, zbuf = refs[k:k + 4]
        k += 4
        if with_deg:
            ones_v, zdeg_v = refs[k:k + 2]
            k += 2
        agg_sp = refs[k]
        k += 1
        if with_deg:
            deg_sp = refs[k]
            k += 1
        gsem = refs[k]

        c = lax.axis_index("c")
        s = lax.axis_index("s")
        w = c * NS + s

        pltpu.sync_copy(srcw_hbm.at[w], src_v)
        pltpu.sync_copy(dstw_hbm.at[w], dst_v)

        def zrow(r, carry):
            for j in range(CW // LANES):
                zbuf[r, pl.ds(j * LANES, LANES)] = jnp.zeros((LANES,), jnp.float32)
            return carry
        lax.fori_loop(0, GZ, zrow, 0)
        if with_deg:
            def zdrow(r, carry):
                zdeg_v[pl.ds(r * LANES, LANES)] = jnp.zeros((LANES,), jnp.float32)
                return carry
            lax.fori_loop(0, RT // LANES, zdrow, 0)

            def orow(r, carry):
                ones_v[pl.ds(r * LANES, LANES)] = jnp.ones((LANES,), jnp.float32)
                return carry
            lax.fori_loop(0, G // LANES, orow, 0)

        for j in range(P):
            h_hbm = h_hbms[j]
            agg_hbm = agg_hbms[j]
            deg_pass = with_deg and j == 0

            for t in range(RT // GZ):
                pltpu.sync_copy(zbuf, agg_sp.at[pl.ds(s * RT + t * GZ, GZ)])
            if deg_pass:
                pltpu.sync_copy(zdeg_v, deg_sp.at[pl.ds(s * RT, RT)])
            plsc.subcore_barrier()

            pltpu.async_copy(h_hbm.at[src_v.at[0]], gbuf.at[0], gsem)

            def step(i, carry):
                pltpu.async_copy(h_hbm.at[src_v.at[i + 1]],
                                 gbuf.at[lax.rem(i + 1, 2)], gsem)
                pltpu.make_async_copy(h_hbm.at[src_v.at[i]],
                                      gbuf.at[lax.rem(i, 2)], gsem).wait()
                pltpu.sync_copy(gbuf.at[lax.rem(i, 2)],
                                agg_sp.at[dst_v.at[i]], add=True)
                if deg_pass:
                    pltpu.sync_copy(ones_v, deg_sp.at[dst_v.at[i]], add=True)
                return carry
            lax.fori_loop(0, K - 1, step, 0)
            last = K - 1
            pltpu.make_async_copy(h_hbm.at[src_v.at[last]],
                                  gbuf.at[lax.rem(last, 2)], gsem).wait()
            pltpu.sync_copy(gbuf.at[lax.rem(last, 2)],
                            agg_sp.at[dst_v.at[last]], add=True)
            if deg_pass:
                pltpu.sync_copy(ones_v, deg_sp.at[dst_v.at[last]], add=True)

            plsc.subcore_barrier()
            pltpu.sync_copy(agg_sp.at[pl.ds(s * RT, RT)],
                            agg_hbm.at[c, pl.ds(s * RT, RT)])
            if deg_pass:
                pltpu.sync_copy(deg_sp.at[pl.ds(s * RT, RT)],
                                deg_hbm.at[c, pl.ds(s * RT, RT)])

    out_type = [jax.ShapeDtypeStruct((NC, n_pad, CW), jnp.float32)
                for _ in range(P)]
    if with_deg:
        out_type.append(jax.ShapeDtypeStruct((NC, n_pad), jnp.float32))
    scratch = [
        pltpu.VMEM((K, G), jnp.int32),
        pltpu.VMEM((K, G), jnp.int32),
        pltpu.VMEM((2, G, CW), jnp.float32),
        pltpu.VMEM((GZ, CW), jnp.float32),
    ]
    if with_deg:
        scratch.append(pltpu.VMEM((G,), jnp.float32))
        scratch.append(pltpu.VMEM((RT,), jnp.float32))
    scratch.append(pltpu.VMEM_SHARED((n_pad, CW), jnp.float32))
    if with_deg:
        scratch.append(pltpu.VMEM_SHARED((n_pad,), jnp.float32))
    scratch.append(pltpu.SemaphoreType.DMA)

    mesh = plsc.VectorSubcoreMesh(core_axis_name="c", subcore_axis_name="s",
                                  num_cores=NC, num_subcores=NS)
    return pl.kernel(
        body, out_type=out_type, mesh=mesh, scratch_types=scratch,
        compiler_params=pltpu.CompilerParams(use_tc_tiling_on_sc=False))


def _tc_combine(h_slabs, agg_slabs, deg3, wst, wnt, brow, relu, split_out,
                n_pad, d):
    BN = 2048
    grid = (n_pad // BN,)
    nh = len(h_slabs)
    P = len(agg_slabs)

    def body(*refs):
        h_refs = refs[0:nh]
        agg_refs = refs[nh:nh + P]
        deg_ref, ws_ref, wn_ref, b_ref = refs[nh + P:nh + P + 4]
        out_refs = refs[nh + P + 4:]

        if nh == 1:
            h = h_refs[0][...]
        else:
            h = jnp.concatenate([r[...] for r in h_refs], axis=1)
        deg = deg_ref[0]
        for q in range(1, NC):
            deg = deg + deg_ref[q]
        scale = 1.0 / jnp.maximum(deg, 1.0)
        parts = []
        for j in range(P):
            a = agg_refs[j][0]
            for q in range(1, NC):
                a = a + agg_refs[j][q]
            parts.append(a)
        hn = jnp.concatenate(parts, axis=1) * scale
        acc = (jnp.dot(h, ws_ref[...], preferred_element_type=jnp.float32)
               + jnp.dot(hn, wn_ref[...], preferred_element_type=jnp.float32)
               + b_ref[...])
        if relu:
            acc = jnp.maximum(acc, 0.0)
        if split_out:
            for j in range(P):
                out_refs[j][...] = acc[:, j * CW:(j + 1) * CW]
        else:
            out_refs[0][...] = acc

    h_specs = [pl.BlockSpec((BN, a.shape[1]), lambda i: (i, 0)) for a in h_slabs]
    agg_specs = [pl.BlockSpec((NC, BN, CW), lambda i: (0, i, 0))
                 for _ in range(P)]
    rest_specs = [
        pl.BlockSpec((NC, BN, 1), lambda i: (0, i, 0)),
        pl.BlockSpec((d, d), lambda i: (0, 0)),
        pl.BlockSpec((d, d), lambda i: (0, 0)),
        pl.BlockSpec((1, d), lambda i: (0, 0)),
    ]
    if split_out:
        out_shape = [jax.ShapeDtypeStruct((n_pad, CW), jnp.float32)
                     for _ in range(P)]
        out_specs = [pl.BlockSpec((BN, CW), lambda i: (i, 0)) for _ in range(P)]
    else:
        out_shape = [jax.ShapeDtypeStruct((n_pad, d), jnp.float32)]
        out_specs = [pl.BlockSpec((BN, d), lambda i: (i, 0))]

    outs = pl.pallas_call(
        body,
        grid=grid,
        in_specs=h_specs + agg_specs + rest_specs,
        out_specs=out_specs,
        out_shape=out_shape,
    )(*h_slabs, *agg_slabs, deg3, wst, wnt, brow)
    return outs


def kernel(inputs, edge_index, W_self1, W_neigh1, b1, W_self2, W_neigh2, b2):
    n, d = inputs.shape
    e = edge_index.shape[1]
    chunk_rows = NS * GZ
    n_pad = ((n + chunk_rows - 1) // chunk_rows) * chunk_rows
    ew = ((e + NW * G - 1) // (NW * G)) * G
    K = ew // G
    e_pad = ew * NW
    P = d // CW

    src = edge_index[0]
    dst = edge_index[1]
    if e_pad > e:
        pad_dst = min(n, n_pad - 1)
        src = jnp.concatenate([src, jnp.zeros((e_pad - e,), jnp.int32)])
        dst = jnp.concatenate([dst, jnp.full((e_pad - e,), pad_dst, jnp.int32)])
    srcw = src.reshape(NW, K, G)
    dstw = dst.reshape(NW, K, G)

    h0 = inputs
    if n_pad > n:
        h0 = jnp.concatenate([inputs, jnp.zeros((n_pad - n, d), jnp.float32)])
    x_slabs = [h0[:, j * CW:(j + 1) * CW] for j in range(P)]

    sc_agg_deg = _make_sc_agg(True, n_pad, d, K)
    sc_agg = _make_sc_agg(False, n_pad, d, K)

    *agg1, deg = sc_agg_deg(*x_slabs, srcw, dstw)
    deg3 = deg[..., None]
    h1_slabs = _tc_combine([h0], agg1, deg3, W_self1.T, W_neigh1.T, b1[None, :],
                           True, True, n_pad, d)
    agg2 = sc_agg(*h1_slabs, srcw, dstw)
    (out,) = _tc_combine(h1_slabs, agg2, deg3, W_self2.T, W_neigh2.T,
                         b2[None, :], False, False, n_pad, d)
    return out[:n]


# speedup vs baseline: 6.5000x; 1.3593x over previous
import jax
import jax.numpy as jnp
from jax import lax
from jax.experimental import pallas as pl
from jax.experimental.pallas import tpu as pltpu
from jax.experimental.pallas import tpu_sc as plsc

NC = 2
NS = 16
NW = NC * NS
G = 64
GZ = 32
CW = 64
LANES = 16


def _make_sc_agg(with_deg, n_pad, d, K):
    RT = n_pad // NS
    P = d // CW

    def body(*refs):
        h_hbms = refs[0:P]
        srcw_hbm, dstw_hbm = refs[P], refs[P + 1]
        agg_hbms = refs[P + 2:2 * P + 2]
        k = 2 * P + 2
        if with_deg:
            deg_hbm = refs[k]
            k += 1
        src_v, dst_v, gbuf, zbuf = refs[k:k + 4]
        k += 4
        if with_deg:
            ones_v, zdeg_v = refs[k:k + 2]
            k += 2
        agg_sp = refs[k]
        k += 1
        if with_deg:
            deg_sp = refs[k]
            k += 1
        gsem = refs[k]

        c = lax.axis_index("c")
        s = lax.axis_index("s")
        w = c * NS + s

        pltpu.sync_copy(srcw_hbm.at[w], src_v)
        pltpu.sync_copy(dstw_hbm.at[w], dst_v)

        def zrow(r, carry):
            for j in range(CW // LANES):
                zbuf[r, pl.ds(j * LANES, LANES)] = jnp.zeros((LANES,), jnp.float32)
            return carry
        lax.fori_loop(0, GZ, zrow, 0)
        if with_deg:
            def zdrow(r, carry):
                zdeg_v[pl.ds(r * LANES, LANES)] = jnp.zeros((LANES,), jnp.float32)
                return carry
            lax.fori_loop(0, RT // LANES, zdrow, 0)

            def orow(r, carry):
                ones_v[pl.ds(r * LANES, LANES)] = jnp.ones((LANES,), jnp.float32)
                return carry
            lax.fori_loop(0, G // LANES, orow, 0)

        for j in range(P):
            h_hbm = h_hbms[j]
            agg_hbm = agg_hbms[j]
            deg_pass = with_deg and j == 0

            for t in range(RT // GZ):
                pltpu.sync_copy(zbuf, agg_sp.at[pl.ds(s * RT + t * GZ, GZ)])
            if deg_pass:
                pltpu.sync_copy(zdeg_v, deg_sp.at[pl.ds(s * RT, RT)])
            plsc.subcore_barrier()

            pltpu.async_copy(h_hbm.at[src_v.at[0]], gbuf.at[0], gsem)

            def step(i, carry):
                pltpu.async_copy(h_hbm.at[src_v.at[i + 1]],
                                 gbuf.at[lax.rem(i + 1, 2)], gsem)
                pltpu.make_async_copy(h_hbm.at[src_v.at[i]],
                                      gbuf.at[lax.rem(i, 2)], gsem).wait()
                pltpu.sync_copy(gbuf.at[lax.rem(i, 2)],
                                agg_sp.at[dst_v.at[i]], add=True)
                if deg_pass:
                    pltpu.sync_copy(ones_v, deg_sp.at[dst_v.at[i]], add=True)
                return carry
            lax.fori_loop(0, K - 1, step, 0)
            last = K - 1
            pltpu.make_async_copy(h_hbm.at[src_v.at[last]],
                                  gbuf.at[lax.rem(last, 2)], gsem).wait()
            pltpu.sync_copy(gbuf.at[lax.rem(last, 2)],
                            agg_sp.at[dst_v.at[last]], add=True)
            if deg_pass:
                pltpu.sync_copy(ones_v, deg_sp.at[dst_v.at[last]], add=True)

            plsc.subcore_barrier()
            pltpu.sync_copy(agg_sp.at[pl.ds(s * RT, RT)],
                            agg_hbm.at[c, pl.ds(s * RT, RT)])
            if deg_pass:
                pltpu.sync_copy(deg_sp.at[pl.ds(s * RT, RT)],
                                deg_hbm.at[c, pl.ds(s * RT, RT)])

    out_type = [jax.ShapeDtypeStruct((NC, n_pad, CW), jnp.float32)
                for _ in range(P)]
    if with_deg:
        out_type.append(jax.ShapeDtypeStruct((NC, n_pad), jnp.float32))
    scratch = [
        pltpu.VMEM((K, G), jnp.int32),
        pltpu.VMEM((K, G), jnp.int32),
        pltpu.VMEM((2, G, CW), jnp.float32),
        pltpu.VMEM((GZ, CW), jnp.float32),
    ]
    if with_deg:
        scratch.append(pltpu.VMEM((G,), jnp.float32))
        scratch.append(pltpu.VMEM((RT,), jnp.float32))
    scratch.append(pltpu.VMEM_SHARED((n_pad, CW), jnp.float32))
    if with_deg:
        scratch.append(pltpu.VMEM_SHARED((n_pad,), jnp.float32))
    scratch.append(pltpu.SemaphoreType.DMA)

    mesh = plsc.VectorSubcoreMesh(core_axis_name="c", subcore_axis_name="s",
                                  num_cores=NC, num_subcores=NS)
    return pl.kernel(
        body, out_type=out_type, mesh=mesh, scratch_types=scratch,
        compiler_params=pltpu.CompilerParams(use_tc_tiling_on_sc=False))


def _tc_combine(h_slabs, agg_slabs, deg3, wst, wnt, brow, relu, split_out,
                n_pad, d):
    BN = 2048
    grid = (n_pad // BN,)
    nh = len(h_slabs)
    P = len(agg_slabs)

    def body(*refs):
        h_refs = refs[0:nh]
        agg_refs = refs[nh:nh + P]
        deg_ref, ws_ref, wn_ref, b_ref = refs[nh + P:nh + P + 4]
        out_refs = refs[nh + P + 4:]

        if nh == 1:
            h = h_refs[0][...]
        else:
            h = jnp.concatenate([r[...] for r in h_refs], axis=1)
        deg = deg_ref[0]
        for q in range(1, NC):
            deg = deg + deg_ref[q]
        scale = 1.0 / jnp.maximum(deg, 1.0)
        parts = []
        for j in range(P):
            a = agg_refs[j][0]
            for q in range(1, NC):
                a = a + agg_refs[j][q]
            parts.append(a)
        hn = jnp.concatenate(parts, axis=1) * scale
        acc = (jnp.dot(h, ws_ref[...], preferred_element_type=jnp.float32)
               + jnp.dot(hn, wn_ref[...], preferred_element_type=jnp.float32)
               + b_ref[...])
        if relu:
            acc = jnp.maximum(acc, 0.0)
        if split_out:
            for j in range(P):
                out_refs[j][...] = acc[:, j * CW:(j + 1) * CW]
        else:
            out_refs[0][...] = acc

    h_specs = [pl.BlockSpec((BN, a.shape[1]), lambda i: (i, 0)) for a in h_slabs]
    agg_specs = [pl.BlockSpec((NC, BN, CW), lambda i: (0, i, 0))
                 for _ in range(P)]
    rest_specs = [
        pl.BlockSpec((NC, BN, 1), lambda i: (0, i, 0)),
        pl.BlockSpec((d, d), lambda i: (0, 0)),
        pl.BlockSpec((d, d), lambda i: (0, 0)),
        pl.BlockSpec((1, d), lambda i: (0, 0)),
    ]
    if split_out:
        out_shape = [jax.ShapeDtypeStruct((n_pad, CW), jnp.float32)
                     for _ in range(P)]
        out_specs = [pl.BlockSpec((BN, CW), lambda i: (i, 0)) for _ in range(P)]
    else:
        out_shape = [jax.ShapeDtypeStruct((n_pad, d), jnp.float32)]
        out_specs = [pl.BlockSpec((BN, d), lambda i: (i, 0))]

    outs = pl.pallas_call(
        body,
        grid=grid,
        in_specs=h_specs + agg_specs + rest_specs,
        out_specs=out_specs,
        out_shape=out_shape,
    )(*h_slabs, *agg_slabs, deg3, wst, wnt, brow)
    return outs


def kernel(inputs, edge_index, W_self1, W_neigh1, b1, W_self2, W_neigh2, b2):
    n, d = inputs.shape
    e = edge_index.shape[1]
    chunk_rows = NS * GZ
    n_pad = ((n + chunk_rows - 1) // chunk_rows) * chunk_rows
    ew = ((e + NW * G - 1) // (NW * G)) * G
    K = ew // G
    e_pad = ew * NW
    P = d // CW

    src = edge_index[0]
    dst = edge_index[1]
    if e_pad > e:
        pad_dst = min(n, n_pad - 1)
        src = jnp.concatenate([src, jnp.zeros((e_pad - e,), jnp.int32)])
        dst = jnp.concatenate([dst, jnp.full((e_pad - e,), pad_dst, jnp.int32)])
    srcw = src.reshape(NW, K, G)
    dstw = dst.reshape(NW, K, G)

    h0 = inputs
    if n_pad > n:
        h0 = jnp.concatenate([inputs, jnp.zeros((n_pad - n, d), jnp.float32)])
    x_slabs = [h0[:, j * CW:(j + 1) * CW] for j in range(P)]

    sc_agg_deg = _make_sc_agg(True, n_pad, d, K)
    sc_agg = _make_sc_agg(False, n_pad, d, K)

    *agg1, deg = sc_agg_deg(*x_slabs, srcw, dstw)
    deg3 = deg[..., None]
    h1_slabs = _tc_combine([h0], agg1, deg3, W_self1.T, W_neigh1.T, b1[None, :],
                           True, True, n_pad, d)
    agg2 = sc_agg(*h1_slabs, srcw, dstw)
    (out,) = _tc_combine(h1_slabs, agg2, deg3, W_self2.T, W_neigh2.T,
                         b2[None, :], False, False, n_pad, d)
    return out[:n]
